# Initial kernel scaffold; baseline (speedup 1.0000x reference)
import jax
import jax.numpy as jnp
from jax.experimental import pallas as pl


def kernel(token_ids, weight):
    raise NotImplementedError("write your pallas kernel here")


# speedup vs baseline: 1.7963x; 1.7963x over previous
import functools

import jax
import jax.numpy as jnp
from jax import lax
from jax.experimental import pallas as pl
from jax.experimental.pallas import tpu as pltpu
from jax.experimental.pallas import tpu_sc as plsc

EMBED_DIM = 64
CHUNK = 512


@functools.partial(jax.jit, static_argnames=("b_total",))
def _sc_gather(weight, idx_flat, b_total):
    info = plsc.get_sparse_core_info()
    nw = info.num_cores * info.num_subcores
    b_per_w = b_total // nw
    n_chunks = b_per_w // CHUNK
    mesh = plsc.VectorSubcoreMesh(core_axis_name="c", subcore_axis_name="s")

    @functools.partial(
        pl.kernel,
        mesh=mesh,
        out_type=jax.ShapeDtypeStruct((b_total, EMBED_DIM), jnp.float32),
        compiler_params=pltpu.CompilerParams(use_tc_tiling_on_sc=False),
        scratch_types=[
            pltpu.VMEM((CHUNK,), jnp.int32),
            pltpu.VMEM((CHUNK, EMBED_DIM), jnp.float32),
            pltpu.SemaphoreType.DMA,
        ],
    )
    def k(table_hbm, idx_hbm, out_hbm, idx_v, rows_v, sem):
        wid = lax.axis_index("s") * info.num_cores + lax.axis_index("c")
        w_base = wid * b_per_w

        def body(i, carry):
            base = w_base + i * CHUNK
            pltpu.sync_copy(idx_hbm.at[pl.ds(base, CHUNK)], idx_v)
            pltpu.async_copy(table_hbm.at[idx_v], rows_v, sem).wait()
            pltpu.sync_copy(rows_v, out_hbm.at[pl.ds(base, CHUNK)])
            return carry

        lax.fori_loop(0, n_chunks, body, 0)

    return k(weight, idx_flat)


def kernel(token_ids, weight):
    b, s = token_ids.shape
    b_total = b * s
    idx_flat = token_ids.reshape(b_total).astype(jnp.int32)
    out = _sc_gather(weight, idx_flat, b_total)
    return out.reshape(b, s, EMBED_DIM)


# speedup vs baseline: 1.8719x; 1.0421x over previous
import functools

import jax
import jax.numpy as jnp
from jax import lax
from jax.experimental import pallas as pl
from jax.experimental.pallas import tpu as pltpu
from jax.experimental.pallas import tpu_sc as plsc

EMBED_DIM = 64
CHUNK = 800


@functools.partial(jax.jit, static_argnames=("b_total",))
def _sc_gather(weight, idx_flat, b_total):
    info = plsc.get_sparse_core_info()
    nw = info.num_cores * info.num_subcores
    b_per_w = b_total // nw
    n_chunks = b_per_w // CHUNK
    n_pairs = n_chunks // 2
    mesh = plsc.VectorSubcoreMesh(core_axis_name="c", subcore_axis_name="s")

    @functools.partial(
        pl.kernel,
        mesh=mesh,
        out_type=jax.ShapeDtypeStruct((b_total, EMBED_DIM), jnp.float32),
        compiler_params=pltpu.CompilerParams(use_tc_tiling_on_sc=False),
        scratch_types=[
            pltpu.VMEM((b_per_w,), jnp.int32),
            pltpu.VMEM((CHUNK, EMBED_DIM), jnp.float32),
            pltpu.VMEM((CHUNK, EMBED_DIM), jnp.float32),
            pltpu.SemaphoreType.DMA,
            pltpu.SemaphoreType.DMA,
            pltpu.SemaphoreType.DMA,
            pltpu.SemaphoreType.DMA,
        ],
    )
    def k(table_hbm, idx_hbm, out_hbm, idx_v, rows0, rows1, sg0, sg1, ss0, ss1):
        rows = (rows0, rows1)
        sg = (sg0, sg1)
        ss = (ss0, ss1)
        wid = lax.axis_index("s") * info.num_cores + lax.axis_index("c")
        w_base = wid * b_per_w

        pltpu.sync_copy(idx_hbm.at[pl.ds(w_base, b_per_w)], idx_v)

        def gather_start(chunk_off, b):
            pltpu.async_copy(
                table_hbm.at[idx_v.at[pl.ds(chunk_off, CHUNK)]], rows[b], sg[b]
            )

        def gather_wait(b):
            pltpu.make_async_copy(
                table_hbm.at[idx_v.at[pl.ds(0, CHUNK)]], rows[b], sg[b]
            ).wait()

        def store_start(chunk_off, b):
            pltpu.async_copy(
                rows[b], out_hbm.at[pl.ds(w_base + chunk_off, CHUNK)], ss[b]
            )

        def store_wait(b):
            pltpu.make_async_copy(
                rows[b], out_hbm.at[pl.ds(w_base, CHUNK)], ss[b]
            ).wait()

        for b in range(2):
            gather_start(b * CHUNK, b)

        def pair(g, carry):
            base = g * (2 * CHUNK)
            for b in range(2):
                off = base + b * CHUNK
                gather_wait(b)
                store_start(off, b)
                store_wait(b)
                gather_start(off + 2 * CHUNK, b)
            return carry

        lax.fori_loop(0, n_pairs - 1, pair, 0, unroll=False)

        tail = (n_chunks - 2) * CHUNK
        for b in range(2):
            gather_wait(b)
            store_start(tail + b * CHUNK, b)
        for b in range(2):
            store_wait(b)

    return k(weight, idx_flat)


def kernel(token_ids, weight):
    b, s = token_ids.shape
    b_total = b * s
    idx_flat = token_ids.reshape(b_total).astype(jnp.int32)
    out = _sc_gather(weight, idx_flat, b_total)
    return out.reshape(b, s, EMBED_DIM)
